# Initial kernel scaffold; baseline (speedup 1.0000x reference)
import functools

import jax
import jax.numpy as jnp
from jax import lax
from jax.experimental import pallas as pl
from jax.experimental.pallas import tpu as pltpu
from jax.experimental.pallas import tpu_sc as plsc

N = 10000
E = 320000
DIN = 128
DH = 256
DOUT = 128
G = 64

NUM_TILES = 16
CHUNK = 128
NPAD = 10240
ROWS_PER_TILE = NPAD // NUM_TILES
EPT = -(-E // (NUM_TILES * CHUNK)) * CHUNK
EPAD = EPT * NUM_TILES
NCHUNK = EPT // CHUNK
BM = 512
NBLK = NPAD // BM


def _make_sc_agg(dc):
    mesh = plsc.VectorSubcoreMesh(core_axis_name="c", subcore_axis_name="s")

    @functools.partial(
        pl.kernel,
        out_type=jax.ShapeDtypeStruct((NPAD, 2 * dc), jnp.float32),
        mesh=mesh,
        scratch_types=[
            pltpu.VMEM((CHUNK,), jnp.int32),
            pltpu.VMEM((CHUNK,), jnp.int32),
            pltpu.VMEM((CHUNK, dc), jnp.float32),
            pltpu.VMEM_SHARED((NPAD, dc), jnp.float32),
            pltpu.SemaphoreType.DMA,
        ],
    )
    def sc_agg(table, srcp, dst, z, idx_s, idx_d, rows, acc, sem):
        c = lax.axis_index("c")
        s = lax.axis_index("s")
        r0 = s * ROWS_PER_TILE
        pltpu.sync_copy(table.at[pl.ds(c * NPAD + r0, ROWS_PER_TILE)],
                        acc.at[pl.ds(r0, ROWS_PER_TILE)])
        plsc.subcore_barrier()
        ebase = s * EPT

        def chunk_body(j, carry):
            off = ebase + j * CHUNK
            pltpu.sync_copy(srcp.at[pl.ds(c * EPAD + off, CHUNK)], idx_s)
            pltpu.sync_copy(dst.at[pl.ds(off, CHUNK)], idx_d)
            pltpu.async_copy(table.at[idx_s], rows, sem).wait()
            pltpu.sync_copy(rows, acc.at[idx_d], add=True)
            return carry

        lax.fori_loop(0, NCHUNK, chunk_body, 0)
        plsc.subcore_barrier()
        pltpu.sync_copy(acc.at[pl.ds(r0, ROWS_PER_TILE)],
                        z.at[pl.ds(r0, ROWS_PER_TILE), pl.ds(c * dc, dc)])

    return sc_agg


_sc_agg_l1 = _make_sc_agg(DIN // 2)
_sc_agg_l2 = _make_sc_agg(DH // 2)


def _mm1_body(z_ref, w_ref, b_ref, out_ref):
    h = jnp.dot(z_ref[...], w_ref[...], preferred_element_type=jnp.float32)
    h = jnp.maximum(h + b_ref[...], 0.0)
    out_ref[0] = h[:, : DH // 2]
    out_ref[1] = h[:, DH // 2:]


def _tc_mm1(z, w, b):
    return pl.pallas_call(
        _mm1_body,
        grid=(NBLK,),
        in_specs=[
            pl.BlockSpec((BM, DIN), lambda i: (i, 0)),
            pl.BlockSpec((DIN, DH), lambda i: (0, 0)),
            pl.BlockSpec((1, DH), lambda i: (0, 0)),
        ],
        out_specs=pl.BlockSpec((2, BM, DH // 2), lambda i: (0, i, 0)),
        out_shape=jax.ShapeDtypeStruct((2, NPAD, DH // 2), jnp.float32),
    )(z, w, b)


def _mm2_body(z_ref, w2_ref, b2_ref, batch_ref, w3_ref, b3_ref, out_ref,
              acc_ref, cnt_ref):
    i = pl.program_id(0)

    @pl.when(i == 0)
    def _():
        acc_ref[...] = jnp.zeros_like(acc_ref)
        cnt_ref[...] = jnp.zeros_like(cnt_ref)

    h = jnp.dot(z_ref[...], w2_ref[...], preferred_element_type=jnp.float32)
    h = jnp.maximum(h + b2_ref[...], 0.0)
    gid = lax.broadcasted_iota(jnp.int32, (BM, G), 1)
    onehot = (batch_ref[...] == gid).astype(jnp.float32)
    acc_ref[...] += lax.dot_general(
        onehot, h, (((0,), (0,)), ((), ())),
        preferred_element_type=jnp.float32)
    cnt_ref[...] += lax.dot_general(
        onehot, jnp.ones((BM, 1), jnp.float32), (((0,), (0,)), ((), ())),
        preferred_element_type=jnp.float32)

    @pl.when(i == NBLK - 1)
    def _():
        pooled = acc_ref[...] / jnp.maximum(cnt_ref[...], 1.0)
        out_ref[...] = jnp.dot(
            pooled, w3_ref[...], preferred_element_type=jnp.float32
        ) + b3_ref[...]


def _tc_mm2(z, w2, b2, batch2d, w3, b3):
    return pl.pallas_call(
        _mm2_body,
        grid=(NBLK,),
        in_specs=[
            pl.BlockSpec((BM, DH), lambda i: (i, 0)),
            pl.BlockSpec((DH, DH), lambda i: (0, 0)),
            pl.BlockSpec((1, DH), lambda i: (0, 0)),
            pl.BlockSpec((BM, 1), lambda i: (i, 0)),
            pl.BlockSpec((DH, DOUT), lambda i: (0, 0)),
            pl.BlockSpec((1, DOUT), lambda i: (0, 0)),
        ],
        out_specs=pl.BlockSpec((G, DOUT), lambda i: (0, 0)),
        out_shape=jax.ShapeDtypeStruct((G, DOUT), jnp.float32),
        scratch_shapes=[
            pltpu.VMEM((G, DH), jnp.float32),
            pltpu.VMEM((G, 1), jnp.float32),
        ],
    )(z, w2, b2, batch2d, w3, b3)


def kernel(x, edge_index, batch, W1, b1, W2, b2, W3, b3):
    src = edge_index[0]
    dst = edge_index[1]
    pad_e = EPAD - E
    src_p = jnp.concatenate([src, jnp.zeros((pad_e,), jnp.int32)])
    dst_p = jnp.concatenate([dst, jnp.full((pad_e,), N, jnp.int32)])
    srcp2 = jnp.concatenate([src_p, src_p + NPAD])

    xpad = jnp.pad(x, ((0, NPAD - N), (0, 0)))
    x_flat = jnp.concatenate([xpad[:, : DIN // 2], xpad[:, DIN // 2:]], axis=0)

    z1 = _sc_agg_l1(x_flat, srcp2, dst_p)
    h1 = _tc_mm1(z1, W1, b1.reshape(1, DH))
    z2 = _sc_agg_l2(h1.reshape(2 * NPAD, DH // 2), srcp2, dst_p)

    batch_p = jnp.pad(batch, (0, NPAD - N), constant_values=G)
    return _tc_mm2(z2, W2, b2.reshape(1, DH), batch_p.reshape(NPAD, 1),
                   W3, b3.reshape(1, DOUT))


# speedup vs baseline: 3.5052x; 3.5052x over previous
import functools

import jax
import jax.numpy as jnp
from jax import lax
from jax.experimental import pallas as pl
from jax.experimental.pallas import tpu as pltpu
from jax.experimental.pallas import tpu_sc as plsc

N = 10000
E = 320000
DIN = 128
DH = 256
DOUT = 128
G = 64

NUM_TILES = 16
NUM_CORES = 2
CHUNK = 128
NPAD = 10240
RPT = NPAD // NUM_TILES
EPAD = -(-E // (32 * CHUNK)) * (32 * CHUNK)
EPT1 = EPAD // 32
NCHUNK1 = EPT1 // CHUNK
EPT2 = EPAD // NUM_TILES
NCHUNK2 = EPT2 // CHUNK
BM = 512
NBLK = NPAD // BM

_MESH = plsc.VectorSubcoreMesh(core_axis_name="c", subcore_axis_name="s")


@functools.partial(
    pl.kernel,
    out_type=jax.ShapeDtypeStruct((2, NPAD, DIN), jnp.float32),
    mesh=_MESH,
    scratch_types=[
        pltpu.VMEM((CHUNK,), jnp.int32),
        pltpu.VMEM((CHUNK,), jnp.int32),
        pltpu.VMEM((CHUNK, DIN), jnp.float32),
        pltpu.VMEM_SHARED((NPAD, DIN), jnp.float32),
        pltpu.SemaphoreType.DMA,
    ],
)
def _sc_agg1(table, src, dst, zeros, z, idx_s, idx_d, rows, acc, sem):
    c = lax.axis_index("c")
    s = lax.axis_index("s")
    r0 = s * RPT

    @pl.when(c == 0)
    def _():
        pltpu.sync_copy(table.at[pl.ds(r0, RPT)], acc.at[pl.ds(r0, RPT)])

    @pl.when(c == 1)
    def _():
        pltpu.sync_copy(zeros, acc.at[pl.ds(r0, RPT)])

    plsc.subcore_barrier()
    ebase = (c * NUM_TILES + s) * EPT1

    def chunk_body(j, carry):
        off = ebase + j * CHUNK
        pltpu.sync_copy(src.at[pl.ds(off, CHUNK)], idx_s)
        pltpu.sync_copy(dst.at[pl.ds(off, CHUNK)], idx_d)
        pltpu.async_copy(table.at[idx_s], rows, sem).wait()
        pltpu.sync_copy(rows, acc.at[idx_d], add=True)
        return carry

    lax.fori_loop(0, NCHUNK1, chunk_body, 0)
    plsc.subcore_barrier()
    pltpu.sync_copy(acc.at[pl.ds(r0, RPT)], z.at[c, pl.ds(r0, RPT)])


@functools.partial(
    pl.kernel,
    out_type=jax.ShapeDtypeStruct((2, NPAD, DH // 2), jnp.float32),
    mesh=_MESH,
    scratch_types=[
        pltpu.VMEM((CHUNK,), jnp.int32),
        pltpu.VMEM((CHUNK,), jnp.int32),
        pltpu.VMEM((CHUNK, DH // 2), jnp.float32),
        pltpu.VMEM_SHARED((NPAD, DH // 2), jnp.float32),
        pltpu.SemaphoreType.DMA,
    ],
)
def _sc_agg2(table, srcp, dst, z, idx_s, idx_d, rows, acc, sem):
    c = lax.axis_index("c")
    s = lax.axis_index("s")
    r0 = s * RPT
    pltpu.sync_copy(table.at[pl.ds(c * NPAD + r0, RPT)],
                    acc.at[pl.ds(r0, RPT)])
    plsc.subcore_barrier()
    ebase = s * EPT2

    def chunk_body(j, carry):
        off = ebase + j * CHUNK
        pltpu.sync_copy(srcp.at[pl.ds(c * EPAD + off, CHUNK)], idx_s)
        pltpu.sync_copy(dst.at[pl.ds(off, CHUNK)], idx_d)
        pltpu.async_copy(table.at[idx_s], rows, sem).wait()
        pltpu.sync_copy(rows, acc.at[idx_d], add=True)
        return carry

    lax.fori_loop(0, NCHUNK2, chunk_body, 0)
    plsc.subcore_barrier()
    pltpu.sync_copy(acc.at[pl.ds(r0, RPT)], z.at[c, pl.ds(r0, RPT)])


def _mm1_body(z_ref, w_ref, b_ref, out_ref):
    zsum = z_ref[0] + z_ref[1]
    h = jnp.dot(zsum, w_ref[...], preferred_element_type=jnp.float32)
    h = jnp.maximum(h + b_ref[...], 0.0)
    out_ref[0] = h[:, : DH // 2]
    out_ref[1] = h[:, DH // 2:]


def _tc_mm1(z, w, b):
    return pl.pallas_call(
        _mm1_body,
        grid=(NBLK,),
        in_specs=[
            pl.BlockSpec((2, BM, DIN), lambda i: (0, i, 0)),
            pl.BlockSpec((DIN, DH), lambda i: (0, 0)),
            pl.BlockSpec((1, DH), lambda i: (0, 0)),
        ],
        out_specs=pl.BlockSpec((2, BM, DH // 2), lambda i: (0, i, 0)),
        out_shape=jax.ShapeDtypeStruct((2, NPAD, DH // 2), jnp.float32),
    )(z, w, b)


def _mm2_body(z_ref, w2_ref, b2_ref, batch_ref, w3_ref, b3_ref, out_ref,
              acc_ref, cnt_ref):
    i = pl.program_id(0)

    @pl.when(i == 0)
    def _():
        acc_ref[...] = jnp.zeros_like(acc_ref)
        cnt_ref[...] = jnp.zeros_like(cnt_ref)

    h = jnp.dot(z_ref[0], w2_ref[...][: DH // 2],
                preferred_element_type=jnp.float32)
    h += jnp.dot(z_ref[1], w2_ref[...][DH // 2:],
                 preferred_element_type=jnp.float32)
    h = jnp.maximum(h + b2_ref[...], 0.0)
    gid = lax.broadcasted_iota(jnp.int32, (BM, G), 1)
    onehot = (batch_ref[...] == gid).astype(jnp.float32)
    acc_ref[...] += lax.dot_general(
        onehot, h, (((0,), (0,)), ((), ())),
        preferred_element_type=jnp.float32)
    cnt_ref[...] += lax.dot_general(
        onehot, jnp.ones((BM, 1), jnp.float32), (((0,), (0,)), ((), ())),
        preferred_element_type=jnp.float32)

    @pl.when(i == NBLK - 1)
    def _():
        pooled = acc_ref[...] / jnp.maximum(cnt_ref[...], 1.0)
        out_ref[...] = jnp.dot(
            pooled, w3_ref[...], preferred_element_type=jnp.float32
        ) + b3_ref[...]


def _tc_mm2(z, w2, b2, batch2d, w3, b3):
    return pl.pallas_call(
        _mm2_body,
        grid=(NBLK,),
        in_specs=[
            pl.BlockSpec((2, BM, DH // 2), lambda i: (0, i, 0)),
            pl.BlockSpec((DH, DH), lambda i: (0, 0)),
            pl.BlockSpec((1, DH), lambda i: (0, 0)),
            pl.BlockSpec((BM, 1), lambda i: (i, 0)),
            pl.BlockSpec((DH, DOUT), lambda i: (0, 0)),
            pl.BlockSpec((1, DOUT), lambda i: (0, 0)),
        ],
        out_specs=pl.BlockSpec((G, DOUT), lambda i: (0, 0)),
        out_shape=jax.ShapeDtypeStruct((G, DOUT), jnp.float32),
        scratch_shapes=[
            pltpu.VMEM((G, DH), jnp.float32),
            pltpu.VMEM((G, 1), jnp.float32),
        ],
    )(z, w2, b2, batch2d, w3, b3)


def kernel(x, edge_index, batch, W1, b1, W2, b2, W3, b3):
    src = edge_index[0]
    dst = edge_index[1]
    pad_e = EPAD - E
    src_p = jnp.concatenate([src, jnp.zeros((pad_e,), jnp.int32)])
    dst_p = jnp.concatenate([dst, jnp.full((pad_e,), N, jnp.int32)])
    srcp2 = jnp.concatenate([src_p, src_p + NPAD])

    xpad = jnp.pad(x, ((0, NPAD - N), (0, 0)))
    zeros_tile = jnp.zeros((RPT, DIN), jnp.float32)

    z1 = _sc_agg1(xpad, src_p, dst_p, zeros_tile)
    h1 = _tc_mm1(z1, W1, b1.reshape(1, DH))
    z2 = _sc_agg2(h1.reshape(2 * NPAD, DH // 2), srcp2, dst_p)

    batch_p = jnp.pad(batch, (0, NPAD - N), constant_values=G)
    return _tc_mm2(z2, W2, b2.reshape(1, DH), batch_p.reshape(NPAD, 1),
                   W3, b3.reshape(1, DOUT))
